# baseline (speedup 1.0000x reference)
import jax
import jax.numpy as jnp
from jax import lax
from jax.experimental import pallas as pl
from jax.experimental.pallas import tpu as pltpu
from jax.experimental.pallas import tpu_sc as plsc

EMB_DIM = 64
BATCH = 4096
LCTX = 20
NNEG = 5
NWORK = 32
LANES = 16

P_TOT = BATCH * LCTX
P_PER_W = P_TOT // NWORK
R_STEP = 8
P_CHUNK = R_STEP * LCTX
P_STEPS = P_PER_W // P_CHUNK

B_PER_W = BATCH // NWORK
N_STEPS = B_PER_W // R_STEP
NV_CHUNK = R_STEP * NNEG

LUT_PAD = 1232

OFF_GU = 0
OFF_GV = P_TOT
OFF_GNU = 2 * P_TOT
OFF_GNV = 3 * P_TOT
OUT_ROWS = 3 * P_TOT + BATCH * NNEG


def _lane_sum(v):
    for sh in (8, 4, 2, 1):
        v = v + jnp.take(v, lax.iota(jnp.int32, LANES) ^ sh)
    return v


def _score_to_idx(score):
    s = jnp.minimum(jnp.maximum(score, -6.0), 6.0)
    return ((s + 6.01) * 100.0).astype(jnp.int32)


def _fetch_idx_rows(idx_h, rb, width, dst_v, sem):
    def cp(r, c):
        pltpu.async_copy(idx_h.at[rb + r, :], dst_v.at[r, pl.ds(0, width)],
                         sem)
        return c
    lax.fori_loop(0, R_STEP, cp, 0)


def _drain_words(hbm_1d, vmem_1d, nwords, sem):
    pltpu.make_async_copy(hbm_1d.at[pl.ds(0, nwords)],
                          vmem_1d.at[pl.ds(0, nwords)], sem).wait()


def _issue_20(table, i20_v, dst_v, sem):
    def issue(r, c):
        iv0 = i20_v[r, pl.ds(0, LANES)]
        iv1 = i20_v[r, pl.ds(LANES, LANES)]
        base = r * LCTX
        for j in range(LANES):
            pltpu.async_copy(table.at[pl.ds(iv0[j] * EMB_DIM, EMB_DIM)],
                             dst_v.at[pl.ds((base + j) * EMB_DIM, EMB_DIM)],
                             sem)
        for j in range(LCTX - LANES):
            pltpu.async_copy(table.at[pl.ds(iv1[j] * EMB_DIM, EMB_DIM)],
                             dst_v.at[pl.ds((base + LANES + j) * EMB_DIM,
                                            EMB_DIM)], sem)
        return c
    lax.fori_loop(0, R_STEP, issue, 0)


def _sc_body(u_w, v_w, lutp, lutn, pos_u, pos_v, neg_u, neg_v, out,
             lutp_v, lutn_v, iu_v, iv_v, embu_v, embv_v, gu_v, gv_v,
             inv_v, embnv_v, gnv_v, sem, isem):
    wid = lax.axis_index("s") * 2 + lax.axis_index("c")

    pltpu.sync_copy(lutp, lutp_v)
    pltpu.sync_copy(lutn, lutn_v)

    def pos_step(s, carry):
        rb = wid * (B_PER_W) + s * R_STEP
        off = wid * P_PER_W + s * P_CHUNK
        _fetch_idx_rows(pos_u, rb, LCTX, iu_v, isem)
        _fetch_idx_rows(pos_v, rb, LCTX, iv_v, isem)
        _drain_words(lutp, lutp_v, 2 * R_STEP * LCTX, isem)
        _issue_20(u_w, iu_v, embu_v, sem)
        _issue_20(v_w, iv_v, embv_v, sem)
        pltpu.make_async_copy(u_w.at[pl.ds(0, P_CHUNK * EMB_DIM)], embu_v,
                              sem).wait()
        pltpu.make_async_copy(u_w.at[pl.ds(0, P_CHUNK * EMB_DIM)], embv_v,
                              sem).wait()

        def prow(r, carry2):
            uu = [embu_v[pl.ds(r * EMB_DIM + k * LANES, LANES)]
                  for k in range(4)]
            vv = [embv_v[pl.ds(r * EMB_DIM + k * LANES, LANES)]
                  for k in range(4)]
            prod = uu[0] * vv[0]
            for k in range(1, 4):
                prod = prod + uu[k] * vv[k]
            score = _lane_sum(prod)[0]
            coef = lutp_v[pl.ds(_score_to_idx(score), LANES)][0]
            for k in range(4):
                gv_v[r, pl.ds(k * LANES, LANES)] = coef * uu[k]
                gu_v[r, pl.ds(k * LANES, LANES)] = coef * vv[k]
            return carry2

        lax.fori_loop(0, P_CHUNK, prow, 0)
        pltpu.sync_copy(gu_v, out.at[pl.ds(OFF_GU + off, P_CHUNK)])
        pltpu.sync_copy(gv_v, out.at[pl.ds(OFF_GV + off, P_CHUNK)])
        return carry

    lax.fori_loop(0, P_STEPS, pos_step, 0)

    def neg_step(s, carry):
        rb = wid * B_PER_W + s * R_STEP
        offu = rb * LCTX
        offv = rb * NNEG
        _fetch_idx_rows(neg_u, rb, LCTX, iu_v, isem)
        _fetch_idx_rows(neg_v, rb, NNEG, inv_v, isem)
        _drain_words(lutp, lutp_v, R_STEP * (LCTX + NNEG), isem)
        _issue_20(u_w, iu_v, embu_v, sem)

        def issue5(r, c):
            iv5 = inv_v[r, pl.ds(0, LANES)]
            for j in range(NNEG):
                pltpu.async_copy(
                    v_w.at[pl.ds(iv5[j] * EMB_DIM, EMB_DIM)],
                    embnv_v.at[pl.ds((r * NNEG + j) * EMB_DIM, EMB_DIM)],
                    sem)
            return c
        lax.fori_loop(0, R_STEP, issue5, 0)
        pltpu.make_async_copy(u_w.at[pl.ds(0, P_CHUNK * EMB_DIM)], embu_v,
                              sem).wait()
        pltpu.make_async_copy(u_w.at[pl.ds(0, NV_CHUNK * EMB_DIM)], embnv_v,
                              sem).wait()

        def item(b, carry2):
            urow0 = b * LCTX
            vrow0 = b * NNEG
            vv = [[embnv_v[pl.ds((vrow0 + n) * EMB_DIM + k * LANES, LANES)]
                   for k in range(4)] for n in range(NNEG)]

            def lrow(l, gnv_acc):
                uu = [embu_v[pl.ds((urow0 + l) * EMB_DIM + k * LANES,
                             LANES)] for k in range(4)]
                gnu_row = None
                new_acc = []
                for n in range(NNEG):
                    prod = uu[0] * vv[n][0]
                    for k in range(1, 4):
                        prod = prod + uu[k] * vv[n][k]
                    score = _lane_sum(prod)[0]
                    coef = lutn_v[pl.ds(_score_to_idx(score), LANES)][0]
                    if gnu_row is None:
                        gnu_row = [coef * vv[n][k] for k in range(4)]
                    else:
                        gnu_row = [gnu_row[k] + coef * vv[n][k]
                                   for k in range(4)]
                    new_acc.append(tuple(gnv_acc[n][k] + coef * uu[k]
                                         for k in range(4)))
                for k in range(4):
                    gu_v[urow0 + l, pl.ds(k * LANES, LANES)] = gnu_row[k]
                return tuple(new_acc)

            zeros = jnp.zeros((LANES,), jnp.float32)
            init = tuple(tuple(zeros for _ in range(4)) for _ in range(NNEG))
            gnv_acc = lax.fori_loop(0, LCTX, lrow, init)
            for n in range(NNEG):
                for k in range(4):
                    gnv_v[vrow0 + n, pl.ds(k * LANES, LANES)] = gnv_acc[n][k]
            return carry2

        lax.fori_loop(0, R_STEP, item, 0)
        pltpu.sync_copy(gu_v, out.at[pl.ds(OFF_GNU + offu, P_CHUNK)])
        pltpu.sync_copy(gnv_v, out.at[pl.ds(OFF_GNV + offv, NV_CHUNK)])
        return carry

    lax.fori_loop(0, N_STEPS, neg_step, 0)


@jax.jit
def _run(u_weight, v_weight, lutp, lutn, pos_u, pos_v, neg_u, neg_v):
    mesh = plsc.VectorSubcoreMesh(core_axis_name="c", subcore_axis_name="s",
                                  num_cores=2, num_subcores=16)
    f = pl.kernel(
        _sc_body,
        out_type=jax.ShapeDtypeStruct((OUT_ROWS, EMB_DIM), jnp.float32),
        mesh=mesh,
        scratch_types=[
            pltpu.VMEM((LUT_PAD,), jnp.float32),
            pltpu.VMEM((LUT_PAD,), jnp.float32),
            pltpu.VMEM((R_STEP, 2 * LANES), jnp.int32),
            pltpu.VMEM((R_STEP, 2 * LANES), jnp.int32),
            pltpu.VMEM((P_CHUNK * EMB_DIM,), jnp.float32),
            pltpu.VMEM((P_CHUNK * EMB_DIM,), jnp.float32),
            pltpu.VMEM((P_CHUNK, EMB_DIM), jnp.float32),
            pltpu.VMEM((P_CHUNK, EMB_DIM), jnp.float32),
            pltpu.VMEM((R_STEP, LANES), jnp.int32),
            pltpu.VMEM((NV_CHUNK * EMB_DIM,), jnp.float32),
            pltpu.VMEM((NV_CHUNK, EMB_DIM), jnp.float32),
            pltpu.SemaphoreType.DMA,
            pltpu.SemaphoreType.DMA,
        ],
    )
    return f(u_weight, v_weight, lutp, lutn, pos_u, pos_v, neg_u, neg_v)


def kernel(u_weight, v_weight, lookup_table, pos_u, pos_v, neg_u, neg_v, lr):
    lutp = (1.0 - lookup_table) * lr
    lutn = -lookup_table * (1.0 * lr)
    pad = LUT_PAD - lutp.shape[0]
    lutp = jnp.pad(lutp.astype(jnp.float32), (0, pad))
    lutn = jnp.pad(lutn.astype(jnp.float32), (0, pad))
    return _run(
        u_weight.reshape(-1), v_weight.reshape(-1), lutp, lutn,
        pos_u.astype(jnp.int32),
        pos_v.astype(jnp.int32),
        neg_u.astype(jnp.int32),
        neg_v.astype(jnp.int32),
    )


# speedup vs baseline: 1.3645x; 1.3645x over previous
import jax
import jax.numpy as jnp
from jax import lax
from jax.experimental import pallas as pl
from jax.experimental.pallas import tpu as pltpu
from jax.experimental.pallas import tpu_sc as plsc

EMB_DIM = 64
BATCH = 4096
LCTX = 20
NNEG = 5
NWORK = 32
LANES = 16

P_TOT = BATCH * LCTX
P_PER_W = P_TOT // NWORK
R_STEP = 8
P_CHUNK = R_STEP * LCTX
P_STEPS = P_PER_W // P_CHUNK

B_PER_W = BATCH // NWORK
N_STEPS = B_PER_W // R_STEP
NV_CHUNK = R_STEP * NNEG

LUT_PAD = 1232

OFF_GU = 0
OFF_GV = P_TOT
OFF_GNU = 2 * P_TOT
OFF_GNV = 3 * P_TOT
OUT_ROWS = 3 * P_TOT + BATCH * NNEG


def _lane_sum(v):
    for sh in (8, 4, 2, 1):
        v = v + jnp.take(v, lax.iota(jnp.int32, LANES) ^ sh)
    return v


def _score_to_idx(score):
    s = jnp.minimum(jnp.maximum(score, -6.0), 6.0)
    return ((s + 6.01) * 100.0).astype(jnp.int32)


def _fetch_idx_rows(idx_h, rb, width, dst_v, sem):
    def cp(r, c):
        pltpu.async_copy(idx_h.at[rb + r, :], dst_v.at[r, pl.ds(0, width)],
                         sem)
        return c
    lax.fori_loop(0, R_STEP, cp, 0)


def _drain_words(hbm_1d, vmem_1d, nwords, sem):
    pltpu.make_async_copy(hbm_1d.at[pl.ds(0, nwords)],
                          vmem_1d.at[pl.ds(0, nwords)], sem).wait()


def _issue_20(table, i20_v, dst_v, sem):
    def issue(r, c):
        iv0 = i20_v[r, pl.ds(0, LANES)]
        iv1 = i20_v[r, pl.ds(LANES, LANES)]
        base = r * LCTX
        for j in range(LANES):
            pltpu.async_copy(table.at[iv0[j]], dst_v.at[base + j], sem)
        for j in range(LCTX - LANES):
            pltpu.async_copy(table.at[iv1[j]], dst_v.at[base + LANES + j],
                             sem)
        return c
    lax.fori_loop(0, R_STEP, issue, 0)


def _issue_5(table, i5_v, dst_v, sem):
    def issue(r, c):
        iv5 = i5_v[r, pl.ds(0, LANES)]
        for j in range(NNEG):
            pltpu.async_copy(table.at[iv5[j]], dst_v.at[r * NNEG + j], sem)
        return c
    lax.fori_loop(0, R_STEP, issue, 0)


def _sc_body(u_w, v_w, lutp, lutn, pos_u, pos_v, neg_u, neg_v, out,
             lutp_v, lutn_v,
             iu_a, iv_a, iu_b, iv_b,
             embu_a, embv_a, embu_b, embv_b,
             inv_a, inv_b, embnv_a, embnv_b,
             sem_a, sem_b, isem):
    wid = lax.axis_index("s") * 2 + lax.axis_index("c")

    pltpu.sync_copy(lutp, lutp_v)
    pltpu.sync_copy(lutn, lutn_v)

    max_rb = BATCH - R_STEP

    def step_rb(s):
        return jnp.minimum(wid * B_PER_W + s * R_STEP, max_rb)

    def pos_fetch(s, iu_v, iv_v, embu_v, embv_v, sem):
        rb = step_rb(s)
        _fetch_idx_rows(pos_u, rb, LCTX, iu_v, isem)
        _fetch_idx_rows(pos_v, rb, LCTX, iv_v, isem)
        _drain_words(lutp, lutp_v, 2 * R_STEP * LCTX, isem)
        _issue_20(u_w, iu_v, embu_v, sem)
        _issue_20(v_w, iv_v, embv_v, sem)

    def pos_compute(s, embu_v, embv_v, sem):
        off = wid * P_PER_W + s * P_CHUNK
        pltpu.make_async_copy(u_w.at[pl.ds(0, P_CHUNK), :], embu_v,
                              sem).wait()
        pltpu.make_async_copy(u_w.at[pl.ds(0, P_CHUNK), :], embv_v,
                              sem).wait()

        def prow(r, carry2):
            uu = [embu_v[r, pl.ds(k * LANES, LANES)] for k in range(4)]
            vv = [embv_v[r, pl.ds(k * LANES, LANES)] for k in range(4)]
            prod = uu[0] * vv[0]
            for k in range(1, 4):
                prod = prod + uu[k] * vv[k]
            score = _lane_sum(prod)[0]
            coef = lutp_v[pl.ds(_score_to_idx(score), LANES)][0]
            for k in range(4):
                embu_v[r, pl.ds(k * LANES, LANES)] = coef * uu[k]
                embv_v[r, pl.ds(k * LANES, LANES)] = coef * vv[k]
            return carry2

        lax.fori_loop(0, P_CHUNK, prow, 0)
        pltpu.sync_copy(embv_v, out.at[pl.ds(OFF_GU + off, P_CHUNK)])
        pltpu.sync_copy(embu_v, out.at[pl.ds(OFF_GV + off, P_CHUNK)])

    pos_fetch(0, iu_a, iv_a, embu_a, embv_a, sem_a)

    def pos_pair(s2, carry):
        s = s2 * 2
        pos_fetch(s + 1, iu_b, iv_b, embu_b, embv_b, sem_b)
        pos_compute(s, embu_a, embv_a, sem_a)
        pos_fetch(s + 2, iu_a, iv_a, embu_a, embv_a, sem_a)
        pos_compute(s + 1, embu_b, embv_b, sem_b)
        return carry

    lax.fori_loop(0, P_STEPS // 2 - 1, pos_pair, 0)
    s_last = P_STEPS - 2
    pos_fetch(s_last + 1, iu_b, iv_b, embu_b, embv_b, sem_b)
    pos_compute(s_last, embu_a, embv_a, sem_a)
    pos_compute(s_last + 1, embu_b, embv_b, sem_b)

    def neg_fetch(s, iu_v, inv_v, embu_v, embnv_v, sem):
        rb = step_rb(s)
        _fetch_idx_rows(neg_u, rb, LCTX, iu_v, isem)
        _fetch_idx_rows(neg_v, rb, NNEG, inv_v, isem)
        _drain_words(lutp, lutp_v, R_STEP * (LCTX + NNEG), isem)
        _issue_20(u_w, iu_v, embu_v, sem)
        _issue_5(v_w, inv_v, embnv_v, sem)

    def neg_compute(s, embu_v, embnv_v, sem):
        rb = wid * B_PER_W + s * R_STEP
        offu = rb * LCTX
        offv = rb * NNEG
        pltpu.make_async_copy(u_w.at[pl.ds(0, P_CHUNK), :], embu_v,
                              sem).wait()
        pltpu.make_async_copy(u_w.at[pl.ds(0, NV_CHUNK), :], embnv_v,
                              sem).wait()

        def item(b, carry2):
            urow0 = b * LCTX
            vrow0 = b * NNEG
            vv = [[embnv_v[vrow0 + n, pl.ds(k * LANES, LANES)]
                   for k in range(4)] for n in range(NNEG)]

            def lrow(l, gnv_acc):
                uu = [embu_v[urow0 + l, pl.ds(k * LANES, LANES)]
                      for k in range(4)]
                gnu_row = None
                new_acc = []
                for n in range(NNEG):
                    prod = uu[0] * vv[n][0]
                    for k in range(1, 4):
                        prod = prod + uu[k] * vv[n][k]
                    score = _lane_sum(prod)[0]
                    coef = lutn_v[pl.ds(_score_to_idx(score), LANES)][0]
                    if gnu_row is None:
                        gnu_row = [coef * vv[n][k] for k in range(4)]
                    else:
                        gnu_row = [gnu_row[k] + coef * vv[n][k]
                                   for k in range(4)]
                    new_acc.append(tuple(gnv_acc[n][k] + coef * uu[k]
                                         for k in range(4)))
                for k in range(4):
                    embu_v[urow0 + l, pl.ds(k * LANES, LANES)] = gnu_row[k]
                return tuple(new_acc)

            zeros = jnp.zeros((LANES,), jnp.float32)
            init = tuple(tuple(zeros for _ in range(4)) for _ in range(NNEG))
            gnv_acc = lax.fori_loop(0, LCTX, lrow, init)
            for n in range(NNEG):
                for k in range(4):
                    embnv_v[vrow0 + n, pl.ds(k * LANES, LANES)] = \
                        gnv_acc[n][k]
            return carry2

        lax.fori_loop(0, R_STEP, item, 0)
        pltpu.sync_copy(embu_v, out.at[pl.ds(OFF_GNU + offu, P_CHUNK)])
        pltpu.sync_copy(embnv_v, out.at[pl.ds(OFF_GNV + offv, NV_CHUNK)])

    neg_fetch(0, iu_a, inv_a, embu_a, embnv_a, sem_a)

    def neg_pair(s2, carry):
        s = s2 * 2
        neg_fetch(s + 1, iu_b, inv_b, embu_b, embnv_b, sem_b)
        neg_compute(s, embu_a, embnv_a, sem_a)
        neg_fetch(s + 2, iu_a, inv_a, embu_a, embnv_a, sem_a)
        neg_compute(s + 1, embu_b, embnv_b, sem_b)
        return carry

    lax.fori_loop(0, N_STEPS // 2 - 1, neg_pair, 0)
    s_last = N_STEPS - 2
    neg_fetch(s_last + 1, iu_b, inv_b, embu_b, embnv_b, sem_b)
    neg_compute(s_last, embu_a, embnv_a, sem_a)
    neg_compute(s_last + 1, embu_b, embnv_b, sem_b)


@jax.jit
def _run(u_weight, v_weight, lutp, lutn, pos_u, pos_v, neg_u, neg_v):
    mesh = plsc.VectorSubcoreMesh(core_axis_name="c", subcore_axis_name="s",
                                  num_cores=2, num_subcores=16)
    f = pl.kernel(
        _sc_body,
        out_type=jax.ShapeDtypeStruct((OUT_ROWS, EMB_DIM), jnp.float32),
        mesh=mesh,
        scratch_types=[
            pltpu.VMEM((LUT_PAD,), jnp.float32),
            pltpu.VMEM((LUT_PAD,), jnp.float32),
            pltpu.VMEM((R_STEP, 2 * LANES), jnp.int32),
            pltpu.VMEM((R_STEP, 2 * LANES), jnp.int32),
            pltpu.VMEM((R_STEP, 2 * LANES), jnp.int32),
            pltpu.VMEM((R_STEP, 2 * LANES), jnp.int32),
            pltpu.VMEM((P_CHUNK, EMB_DIM), jnp.float32),
            pltpu.VMEM((P_CHUNK, EMB_DIM), jnp.float32),
            pltpu.VMEM((P_CHUNK, EMB_DIM), jnp.float32),
            pltpu.VMEM((P_CHUNK, EMB_DIM), jnp.float32),
            pltpu.VMEM((R_STEP, LANES), jnp.int32),
            pltpu.VMEM((R_STEP, LANES), jnp.int32),
            pltpu.VMEM((NV_CHUNK, EMB_DIM), jnp.float32),
            pltpu.VMEM((NV_CHUNK, EMB_DIM), jnp.float32),
            pltpu.SemaphoreType.DMA,
            pltpu.SemaphoreType.DMA,
            pltpu.SemaphoreType.DMA,
        ],
    )
    return f(u_weight, v_weight, lutp, lutn, pos_u, pos_v, neg_u, neg_v)


def kernel(u_weight, v_weight, lookup_table, pos_u, pos_v, neg_u, neg_v, lr):
    lutp = (1.0 - lookup_table) * lr
    lutn = -lookup_table * (1.0 * lr)
    pad = LUT_PAD - lutp.shape[0]
    lutp = jnp.pad(lutp.astype(jnp.float32), (0, pad))
    lutn = jnp.pad(lutn.astype(jnp.float32), (0, pad))
    return _run(
        u_weight, v_weight, lutp, lutn,
        pos_u.astype(jnp.int32),
        pos_v.astype(jnp.int32),
        neg_u.astype(jnp.int32),
        neg_v.astype(jnp.int32),
    )


# speedup vs baseline: 1.4175x; 1.0388x over previous
import jax
import jax.numpy as jnp
from jax import lax
from jax.experimental import pallas as pl
from jax.experimental.pallas import tpu as pltpu
from jax.experimental.pallas import tpu_sc as plsc

EMB_DIM = 64
BATCH = 4096
LCTX = 20
NNEG = 5
NWORK = 32
LANES = 16

P_TOT = BATCH * LCTX
P_PER_W = P_TOT // NWORK
R_STEP = 8
P_CHUNK = R_STEP * LCTX
P_STEPS = P_PER_W // P_CHUNK

B_PER_W = BATCH // NWORK
N_STEPS = B_PER_W // R_STEP
NV_CHUNK = R_STEP * NNEG

LUT_PAD = 1232

OFF_GU = 0
OFF_GV = P_TOT
OFF_GNU = 2 * P_TOT
OFF_GNV = 3 * P_TOT
OUT_ROWS = 3 * P_TOT + BATCH * NNEG


def _lane_sum(v):
    for sh in (8, 4, 2, 1):
        v = v + jnp.take(v, lax.iota(jnp.int32, LANES) ^ sh)
    return v


def _score_to_idx(score):
    s = jnp.minimum(jnp.maximum(score, -6.0), 6.0)
    return ((s + 6.01) * 100.0).astype(jnp.int32)


def _fetch_idx_rows(idx_h, rb, width, dst_v, sem):
    def cp(r, c):
        pltpu.async_copy(idx_h.at[rb + r, :], dst_v.at[r, pl.ds(0, width)],
                         sem)
        return c
    lax.fori_loop(0, R_STEP, cp, 0)


def _drain_words(hbm_1d, vmem_1d, nwords, sem):
    pltpu.make_async_copy(hbm_1d.at[pl.ds(0, nwords)],
                          vmem_1d.at[pl.ds(0, nwords)], sem).wait()


def _issue_20(table, i20_v, dst_v, sem):
    def issue(r, c):
        iv0 = i20_v[r, pl.ds(0, LANES)]
        iv1 = i20_v[r, pl.ds(LANES, LANES)]
        base = r * LCTX
        for j in range(LANES):
            pltpu.async_copy(table.at[iv0[j]], dst_v.at[base + j], sem)
        for j in range(LCTX - LANES):
            pltpu.async_copy(table.at[iv1[j]], dst_v.at[base + LANES + j],
                             sem)
        return c
    lax.fori_loop(0, R_STEP, issue, 0)


def _sc_body(u_w, v_w, lutp, lutn, pos_u, pos_v, neg_u, neg_v, out,
             lutp_v, lutn_v, iu_v, iv_v, embu_v, embv_v, gu_v, gv_v,
             inv_v, embnv_v, gnv_v, sem, isem, osem):
    wid = lax.axis_index("s") * 2 + lax.axis_index("c")

    pltpu.sync_copy(lutp, lutp_v)
    pltpu.sync_copy(lutn, lutn_v)

    def _drain_g160():
        pltpu.make_async_copy(u_w.at[pl.ds(0, P_CHUNK), :], gu_v,
                              osem).wait()

    def _drain_g40():
        pltpu.make_async_copy(u_w.at[pl.ds(0, NV_CHUNK), :], gnv_v,
                              osem).wait()

    def pos_step(s, carry):
        rb = wid * (B_PER_W) + s * R_STEP
        off = wid * P_PER_W + s * P_CHUNK

        @pl.when(s > 0)
        def _():
            _drain_g160()
            _drain_g160()

        _fetch_idx_rows(pos_u, rb, LCTX, iu_v, isem)
        _fetch_idx_rows(pos_v, rb, LCTX, iv_v, isem)
        _drain_words(lutp, lutp_v, 2 * R_STEP * LCTX, isem)
        _issue_20(u_w, iu_v, embu_v, sem)
        _issue_20(v_w, iv_v, embv_v, sem)
        pltpu.make_async_copy(u_w.at[pl.ds(0, P_CHUNK), :], embu_v,
                              sem).wait()
        pltpu.make_async_copy(u_w.at[pl.ds(0, P_CHUNK), :], embv_v,
                              sem).wait()

        def prow(r, carry2):
            uu = [embu_v[r, pl.ds(k * LANES, LANES)] for k in range(4)]
            vv = [embv_v[r, pl.ds(k * LANES, LANES)] for k in range(4)]
            prod = uu[0] * vv[0]
            for k in range(1, 4):
                prod = prod + uu[k] * vv[k]
            score = _lane_sum(prod)[0]
            coef = lutp_v[pl.ds(_score_to_idx(score), LANES)][0]
            for k in range(4):
                gv_v[r, pl.ds(k * LANES, LANES)] = coef * uu[k]
                gu_v[r, pl.ds(k * LANES, LANES)] = coef * vv[k]
            return carry2

        lax.fori_loop(0, P_CHUNK, prow, 0)
        pltpu.async_copy(gu_v, out.at[pl.ds(OFF_GU + off, P_CHUNK)], osem)
        pltpu.async_copy(gv_v, out.at[pl.ds(OFF_GV + off, P_CHUNK)], osem)
        return carry

    lax.fori_loop(0, P_STEPS, pos_step, 0)

    def neg_step(s, carry):
        rb = wid * B_PER_W + s * R_STEP
        offu = rb * LCTX
        offv = rb * NNEG

        @pl.when(s == 0)
        def _():
            _drain_g160()
            _drain_g160()

        @pl.when(s > 0)
        def _():
            _drain_g160()
            _drain_g40()

        _fetch_idx_rows(neg_u, rb, LCTX, iu_v, isem)
        _fetch_idx_rows(neg_v, rb, NNEG, inv_v, isem)
        _drain_words(lutp, lutp_v, R_STEP * (LCTX + NNEG), isem)
        _issue_20(u_w, iu_v, embu_v, sem)

        def issue5(r, c):
            iv5 = inv_v[r, pl.ds(0, LANES)]
            for j in range(NNEG):
                pltpu.async_copy(v_w.at[iv5[j]], embnv_v.at[r * NNEG + j],
                                 sem)
            return c
        lax.fori_loop(0, R_STEP, issue5, 0)
        pltpu.make_async_copy(u_w.at[pl.ds(0, P_CHUNK), :], embu_v,
                              sem).wait()
        pltpu.make_async_copy(u_w.at[pl.ds(0, NV_CHUNK), :], embnv_v,
                              sem).wait()

        def item(b, carry2):
            urow0 = b * LCTX
            vrow0 = b * NNEG
            vv = [[embnv_v[vrow0 + n, pl.ds(k * LANES, LANES)]
                   for k in range(4)] for n in range(NNEG)]

            def lrow(l, gnv_acc):
                uu = [embu_v[urow0 + l, pl.ds(k * LANES, LANES)]
                      for k in range(4)]
                gnu_row = None
                new_acc = []
                for n in range(NNEG):
                    prod = uu[0] * vv[n][0]
                    for k in range(1, 4):
                        prod = prod + uu[k] * vv[n][k]
                    score = _lane_sum(prod)[0]
                    coef = lutn_v[pl.ds(_score_to_idx(score), LANES)][0]
                    if gnu_row is None:
                        gnu_row = [coef * vv[n][k] for k in range(4)]
                    else:
                        gnu_row = [gnu_row[k] + coef * vv[n][k]
                                   for k in range(4)]
                    new_acc.append(tuple(gnv_acc[n][k] + coef * uu[k]
                                         for k in range(4)))
                for k in range(4):
                    gu_v[urow0 + l, pl.ds(k * LANES, LANES)] = gnu_row[k]
                return tuple(new_acc)

            zeros = jnp.zeros((LANES,), jnp.float32)
            init = tuple(tuple(zeros for _ in range(4)) for _ in range(NNEG))
            gnv_acc = lax.fori_loop(0, LCTX, lrow, init)
            for n in range(NNEG):
                for k in range(4):
                    gnv_v[vrow0 + n, pl.ds(k * LANES, LANES)] = gnv_acc[n][k]
            return carry2

        lax.fori_loop(0, R_STEP, item, 0)
        pltpu.async_copy(gu_v, out.at[pl.ds(OFF_GNU + offu, P_CHUNK)], osem)
        pltpu.async_copy(gnv_v, out.at[pl.ds(OFF_GNV + offv, NV_CHUNK)],
                         osem)
        return carry

    lax.fori_loop(0, N_STEPS, neg_step, 0)
    _drain_g160()
    _drain_g40()


@jax.jit
def _run(u_weight, v_weight, lutp, lutn, pos_u, pos_v, neg_u, neg_v):
    mesh = plsc.VectorSubcoreMesh(core_axis_name="c", subcore_axis_name="s",
                                  num_cores=2, num_subcores=16)
    f = pl.kernel(
        _sc_body,
        out_type=jax.ShapeDtypeStruct((OUT_ROWS, EMB_DIM), jnp.float32),
        mesh=mesh,
        scratch_types=[
            pltpu.VMEM((LUT_PAD,), jnp.float32),
            pltpu.VMEM((LUT_PAD,), jnp.float32),
            pltpu.VMEM((R_STEP, 2 * LANES), jnp.int32),
            pltpu.VMEM((R_STEP, 2 * LANES), jnp.int32),
            pltpu.VMEM((P_CHUNK, EMB_DIM), jnp.float32),
            pltpu.VMEM((P_CHUNK, EMB_DIM), jnp.float32),
            pltpu.VMEM((P_CHUNK, EMB_DIM), jnp.float32),
            pltpu.VMEM((P_CHUNK, EMB_DIM), jnp.float32),
            pltpu.VMEM((R_STEP, LANES), jnp.int32),
            pltpu.VMEM((NV_CHUNK, EMB_DIM), jnp.float32),
            pltpu.VMEM((NV_CHUNK, EMB_DIM), jnp.float32),
            pltpu.SemaphoreType.DMA,
            pltpu.SemaphoreType.DMA,
            pltpu.SemaphoreType.DMA,
        ],
    )
    return f(u_weight, v_weight, lutp, lutn, pos_u, pos_v, neg_u, neg_v)


def kernel(u_weight, v_weight, lookup_table, pos_u, pos_v, neg_u, neg_v, lr):
    lutp = (1.0 - lookup_table) * lr
    lutn = -lookup_table * (1.0 * lr)
    pad = LUT_PAD - lutp.shape[0]
    lutp = jnp.pad(lutp.astype(jnp.float32), (0, pad))
    lutn = jnp.pad(lutn.astype(jnp.float32), (0, pad))
    return _run(
        u_weight, v_weight, lutp, lutn,
        pos_u.astype(jnp.int32),
        pos_v.astype(jnp.int32),
        neg_u.astype(jnp.int32),
        neg_v.astype(jnp.int32),
    )
